# Initial kernel scaffold; baseline (speedup 1.0000x reference)
import jax
import jax.numpy as jnp
from jax.experimental import pallas as pl


def kernel(scores):
    raise NotImplementedError("write your pallas kernel here")


# speedup vs baseline: 3.4935x; 3.4935x over previous
import functools

import jax
import jax.numpy as jnp
from jax import lax
from jax.experimental import pallas as pl
from jax.experimental.pallas import tpu as pltpu
from jax.experimental.pallas import tpu_sc as plsc

B = 128
N = 32768
L = 16
GROUP = 8
NGROUP = N // (GROUP * L)
NBUF = 3

_NEG = float("-inf")


def _row_scan(row_ref):
    c0 = jnp.full((L,), _NEG, jnp.float32)
    t0 = jnp.full((L,), _NEG, jnp.float32)

    def scan_body(i, carry):
        c, thrv = carry
        base = i * (GROUP * L)
        xs = [row_ref[pl.ds(base + g * L, L)] for g in range(GROUP)]
        gm = xs[0]
        for x in xs[1:]:
            gm = jnp.maximum(gm, x)

        def merge(cin):
            cc = cin
            for x in xs:
                xd = lax.rev(jnp.sort(x), (0,))
                cc = jnp.sort(jnp.maximum(cc, xd))
            tv = -plsc.cummax(-cc)
            return cc, tv

        return lax.cond(jnp.any(gm > thrv), merge, lambda cc: (cc, thrv), c)

    c, thrv = lax.fori_loop(0, NGROUP, scan_body, (c0, t0))
    rv = plsc.all_reduce_population_count(c == thrv)
    return thrv, rv


def _mask_row(row_ref, thrv, rv):
    ones = jnp.full((L,), 1.0, jnp.float32)
    zeros = jnp.full((L,), 0.0, jnp.float32)
    cnt0 = jnp.zeros((L,), jnp.int32)

    def mask_body(i, cnt):
        base = i * (GROUP * L)
        xs = [row_ref[pl.ds(base + g * L, L)] for g in range(GROUP)]
        ges = [x >= thrv for x in xs]
        gts = [x > thrv for x in xs]
        eq_any = ges[0] != gts[0]
        for g in range(1, GROUP):
            eq_any = eq_any | (ges[g] != gts[g])

        def fast(cin):
            for g in range(GROUP):
                row_ref[pl.ds(base + g * L, L)] = jnp.where(ges[g], ones, zeros)
            return cin

        def exact(cin):
            for g in range(GROUP):
                eq = ges[g] != gts[g]
                eqi = jnp.where(eq, 1, 0).astype(jnp.int32)
                excl = plsc.cumsum(eqi) - eqi
                sel = eq & ((cin + excl) < rv)
                row_ref[pl.ds(base + g * L, L)] = jnp.where(
                    gts[g] | sel, ones, zeros
                )
                cin = cin + plsc.all_reduce_population_count(eq)
            return cin

        return lax.cond(jnp.any(eq_any), exact, fast, cnt)

    lax.fori_loop(0, NGROUP, mask_body, cnt0)


def _make_kernel():
    info = plsc.get_sparse_core_info()
    nc, ns = info.num_cores, info.num_subcores
    nw = nc * ns
    rows_per_w = B // nw

    mesh = plsc.VectorSubcoreMesh(core_axis_name="c", subcore_axis_name="s")

    @functools.partial(
        pl.kernel,
        mesh=mesh,
        out_type=jax.ShapeDtypeStruct((B, N), jnp.float32),
        scratch_types=[
            [pltpu.VMEM((N,), jnp.float32) for _ in range(NBUF)],
            [pltpu.SemaphoreType.DMA for _ in range(NBUF)],
            [pltpu.SemaphoreType.DMA for _ in range(NBUF)],
        ],
        compiler_params=pltpu.CompilerParams(needs_layout_passes=False),
    )
    def topk_mask(scores_hbm, out_hbm, bufs, in_sems, out_sems):
        wid = lax.axis_index("s") * nc + lax.axis_index("c")
        base_row = wid * rows_per_w

        copies_in = [None] * rows_per_w
        copies_out = [None] * rows_per_w
        for r in range(min(NBUF - 1, rows_per_w)):
            copies_in[r] = pltpu.async_copy(
                scores_hbm.at[base_row + r], bufs[r % NBUF], in_sems[r % NBUF]
            )

        for r in range(rows_per_w):
            b = r % NBUF
            copies_in[r].wait()
            nxt = r + NBUF - 1
            if nxt < rows_per_w:
                nb = nxt % NBUF
                if nxt - NBUF >= 0 and copies_out[nxt - NBUF] is not None:
                    copies_out[nxt - NBUF].wait()
                copies_in[nxt] = pltpu.async_copy(
                    scores_hbm.at[base_row + nxt], bufs[nb], in_sems[nb]
                )
            thrv, rv = _row_scan(bufs[b])
            _mask_row(bufs[b], thrv, rv)
            copies_out[r] = pltpu.async_copy(
                bufs[b], out_hbm.at[base_row + r], out_sems[b]
            )
        for r in range(rows_per_w):
            if copies_out[r] is not None and r + NBUF >= rows_per_w:
                copies_out[r].wait()

    return topk_mask


def kernel(scores):
    return _make_kernel()(scores)


# speedup vs baseline: 5.4805x; 1.5688x over previous
import functools

import jax
import jax.numpy as jnp
from jax import lax
from jax.experimental import pallas as pl
from jax.experimental.pallas import tpu as pltpu
from jax.experimental.pallas import tpu_sc as plsc

B = 128
N = 32768
L = 16
GS = 16
NGS = N // (GS * L)
GM = 16
NGM = N // (GM * L)

_NEG = float("-inf")


def _row_scan(row_ref):
    c0 = jnp.full((L,), _NEG, jnp.float32)
    t0 = jnp.full((L,), _NEG, jnp.float32)

    def scan_body(i, carry):
        c, thrv = carry
        base = i * (GS * L)
        xs = [row_ref[pl.ds(base + g * L, L)] for g in range(GS)]
        gm = xs[0]
        for x in xs[1:]:
            gm = jnp.maximum(gm, x)

        def merge(cin):
            cc = cin
            for x in xs:
                xd = lax.rev(jnp.sort(x), (0,))
                cc = jnp.sort(jnp.maximum(cc, xd))
            tv = -plsc.cummax(-cc)
            return cc, tv

        return lax.cond(jnp.any(gm > thrv), merge, lambda cc: (cc, thrv), c)

    c, thrv = lax.fori_loop(0, NGS, scan_body, (c0, t0))
    rv = plsc.all_reduce_population_count(c == thrv)
    return thrv, rv


def _mask_row(val_ref, mask_ref, thrv, rv):
    ones = jnp.full((L,), 1.0, jnp.float32)
    zeros = jnp.full((L,), 0.0, jnp.float32)

    def fast_body(i, acc):
        base = i * (GM * L)
        for g in range(GM):
            sl = pl.ds(base + g * L, L)
            x = val_ref[sl]
            ge = x >= thrv
            gt = x > thrv
            mask_ref[sl] = jnp.where(ge, ones, zeros)
            acc = acc + jnp.where(ge != gt, 1, 0).astype(jnp.int32)
        return acc

    acc = lax.fori_loop(0, NGM, fast_body, jnp.zeros((L,), jnp.int32))
    totv = plsc.cummax(lax.rev(plsc.cumsum(acc), (0,)))

    def fixup(_):
        cnt0 = jnp.zeros((L,), jnp.int32)

        def fix_body(i, cnt):
            base = i * (GM * L)
            for g in range(GM):
                sl = pl.ds(base + g * L, L)
                x = val_ref[sl]
                gt = x > thrv
                eq = (x >= thrv) != gt
                eqi = jnp.where(eq, 1, 0).astype(jnp.int32)
                excl = plsc.cumsum(eqi) - eqi
                sel = eq & ((cnt + excl) < rv)
                mask_ref[sl] = jnp.where(gt | sel, ones, zeros)
                cnt = cnt + plsc.all_reduce_population_count(eq)
            return cnt

        lax.fori_loop(0, NGM, fix_body, cnt0)
        return 0

    lax.cond(jnp.any(totv > rv), fixup, lambda _: 0, 0)


def _make_kernel():
    info = plsc.get_sparse_core_info()
    nc, ns = info.num_cores, info.num_subcores
    nw = nc * ns
    rows_per_w = B // nw

    mesh = plsc.VectorSubcoreMesh(core_axis_name="c", subcore_axis_name="s")

    @functools.partial(
        pl.kernel,
        mesh=mesh,
        out_type=jax.ShapeDtypeStruct((B, N), jnp.float32),
        scratch_types=[
            [pltpu.VMEM((N,), jnp.float32) for _ in range(2)],
            pltpu.VMEM((N,), jnp.float32),
            [pltpu.SemaphoreType.DMA for _ in range(2)],
            pltpu.SemaphoreType.DMA,
        ],
        compiler_params=pltpu.CompilerParams(needs_layout_passes=False),
    )
    def topk_mask(scores_hbm, out_hbm, in_bufs, mask_buf, in_sems, out_sem):
        wid = lax.axis_index("s") * nc + lax.axis_index("c")
        base_row = wid * rows_per_w

        copies_in = [None] * rows_per_w
        copies_out = [None] * rows_per_w
        for r in range(min(2, rows_per_w)):
            copies_in[r] = pltpu.async_copy(
                scores_hbm.at[base_row + r], in_bufs[r % 2], in_sems[r % 2]
            )

        for r in range(rows_per_w):
            b = r % 2
            copies_in[r].wait()
            thrv, rv = _row_scan(in_bufs[b])
            if r > 0:
                copies_out[r - 1].wait()
            _mask_row(in_bufs[b], mask_buf, thrv, rv)
            copies_out[r] = pltpu.async_copy(
                mask_buf, out_hbm.at[base_row + r], out_sem
            )
            if r + 2 < rows_per_w:
                copies_in[r + 2] = pltpu.async_copy(
                    scores_hbm.at[base_row + r + 2], in_bufs[b], in_sems[b]
                )
        copies_out[rows_per_w - 1].wait()

    return topk_mask


def kernel(scores):
    return _make_kernel()(scores)
